# baseline (speedup 1.0000x reference)
import jax
import jax.numpy as jnp
from jax.experimental import pallas as pl
from jax.experimental.pallas import tpu as pltpu

_BT = 128
_HC = 768
_CB = 512
_PBS = 512
_TBS = 512
_SQRT1_2 = 0.7071067811865476


def _route_body(x_ref, wr_ref, dst0_ref, dst1_ref, g1_ref, g2_ref,
                te_ref, lb_ref):
    t = x_ref.shape[0]
    e_num = wr_ref.shape[0]
    nt = te_ref.shape[1]
    x = x_ref[...]
    wr = wr_ref[...]
    logits = jax.lax.dot_general(x, wr, (((1,), (1,)), ((), ())),
                                 preferred_element_type=jnp.float32)
    m = jnp.max(logits, axis=1, keepdims=True)
    ex = jnp.exp(logits - m)
    probs = ex / jnp.sum(ex, axis=1, keepdims=True)
    lane = jax.lax.broadcasted_iota(jnp.int32, (t, e_num), 1)
    p1 = jnp.max(probs, axis=1, keepdims=True)
    i1 = jnp.min(jnp.where(probs == p1, lane, e_num), axis=1, keepdims=True)
    probs2 = jnp.where(lane == i1, -jnp.inf, probs)
    p2 = jnp.max(probs2, axis=1, keepdims=True)
    i2 = jnp.min(jnp.where(probs2 == p2, lane, e_num), axis=1, keepdims=True)
    s = p1 + p2 + 1e-9
    g1_ref[...] = p1 / s
    g2_ref[...] = p2 / s

    one0 = (lane == i1).astype(jnp.float32)
    one1 = (lane == i2).astype(jnp.float32)

    imp = jnp.sum(probs, axis=0, keepdims=True)
    load = jnp.sum(one0 + one1, axis=0, keepdims=True)
    impn = imp / (jnp.sum(imp) + 1e-9)
    loadn = load / (jnp.sum(load) + 1e-9)
    lb_ref[0, 0] = jnp.sum(impn * loadn) * e_num

    cb = _CB
    tri = (jax.lax.broadcasted_iota(jnp.int32, (cb, cb), 0)
           > jax.lax.broadcasted_iota(jnp.int32, (cb, cb), 1)
           ).astype(jnp.float32)
    run = jnp.zeros((1, e_num), jnp.float32)
    ranks = []
    for oh in (one0, one1):
        for b in range(t // cb):
            blk = oh[b * cb:(b + 1) * cb]
            ranks.append(jax.lax.dot_general(
                tri, blk, (((1,), (0,)), ((), ())),
                preferred_element_type=jnp.float32) + run)
            run = run + jnp.sum(blk, axis=0, keepdims=True)
    nb = t // cb
    rank0 = jnp.concatenate(ranks[:nb], axis=0)
    rank1 = jnp.concatenate(ranks[nb:], axis=0)
    counts = run

    ntiles = jnp.floor((counts + (_BT - 1)) * (1.0 / _BT))
    er = jax.lax.broadcasted_iota(jnp.int32, (e_num, e_num), 0)
    ec = jax.lax.broadcasted_iota(jnp.int32, (e_num, e_num), 1)
    ubelow = (er < ec).astype(jnp.float32)
    tile_start = jax.lax.dot_general(
        ntiles, ubelow, (((1,), (0,)), ((), ())),
        preferred_element_type=jnp.float32)
    base = tile_start * float(_BT)
    dst0_ref[...] = jnp.sum(one0 * (base + rank0), axis=1, keepdims=True)
    dst1_ref[...] = jnp.sum(one1 * (base + rank1), axis=1, keepdims=True)

    ident = (er == ec).astype(jnp.float32)
    ts_col = jax.lax.dot_general(
        ident, tile_start, (((1,), (1,)), ((), ())),
        preferred_element_type=jnp.float32)
    tau = jax.lax.broadcasted_iota(jnp.int32, (e_num, nt), 1
                                   ).astype(jnp.float32)
    te = jnp.sum((ts_col <= tau).astype(jnp.float32), axis=0,
                 keepdims=True) - 1.0
    te_ref[...] = te.astype(jnp.int32)


def _gather_body(dst0r_ref, dst1r_ref, g1_ref, g2_ref, x_ref,
                 xs_ref, gs_ref):
    pb = pl.program_id(0)
    t = x_ref.shape[0]
    p_vals = (jax.lax.broadcasted_iota(jnp.int32, (_PBS, t), 0)
              + pb * _PBS).astype(jnp.float32)
    m0 = (dst0r_ref[...] == p_vals)
    m1 = (dst1r_ref[...] == p_vals)
    msum = (m0 | m1).astype(jnp.float32)
    xs_ref[...] = jax.lax.dot_general(
        msum, x_ref[...], (((1,), (0,)), ((), ())),
        preferred_element_type=jnp.float32)
    m0f = m0.astype(jnp.float32)
    m1f = m1.astype(jnp.float32)
    g1 = g1_ref[...]
    g2 = g2_ref[...]
    g1h = g1.astype(jnp.bfloat16).astype(jnp.float32)
    g2h = g2.astype(jnp.bfloat16).astype(jnp.float32)
    gs_ref[...] = (
        jax.lax.dot_general(m0f, g1h, (((1,), (0,)), ((), ())),
                            preferred_element_type=jnp.float32)
        + jax.lax.dot_general(m0f, g1 - g1h, (((1,), (0,)), ((), ())),
                              preferred_element_type=jnp.float32)
        + jax.lax.dot_general(m1f, g2h, (((1,), (0,)), ((), ())),
                              preferred_element_type=jnp.float32)
        + jax.lax.dot_general(m1f, g2 - g2h, (((1,), (0,)), ((), ())),
                              preferred_element_type=jnp.float32))


def _ffn_body(te_ref, xs_ref, gs_ref, w1_ref, w2_ref, ys_ref):
    c = pl.program_id(0)
    i = pl.program_id(1)
    nc = pl.num_programs(0)
    rows = pl.ds(i * _BT, _BT)

    xs = xs_ref[rows, :]
    w1 = w1_ref[0]
    w2 = w2_ref[0]
    h = jax.lax.dot_general(xs, w1, (((1,), (1,)), ((), ())),
                            preferred_element_type=jnp.float32)
    h = 0.5 * h * (1.0 + jax.lax.erf(h * _SQRT1_2))
    y = jax.lax.dot_general(h, w2, (((1,), (1,)), ((), ())),
                            preferred_element_type=jnp.float32)

    @pl.when(c == 0)
    def _init():
        ys_ref[rows, :] = y

    @pl.when(c > 0)
    def _acc():
        ys_ref[rows, :] += y

    @pl.when(c == nc - 1)
    def _scale():
        ys_ref[rows, :] *= gs_ref[rows, :]


def _scatter_body(dst0_ref, dst1_ref, ys_ref, out_ref):
    tb = out_ref.shape[0]
    np_ = ys_ref.shape[0]
    p_row = jax.lax.broadcasted_iota(jnp.int32, (tb, np_), 1
                                     ).astype(jnp.float32)
    msum = ((dst0_ref[...] == p_row) | (dst1_ref[...] == p_row)
            ).astype(jnp.float32)
    out_ref[...] = jax.lax.dot_general(
        msum, ys_ref[...], (((1,), (0,)), ((), ())),
        preferred_element_type=jnp.float32)


def kernel(x, Wr, W1, W2):
    b, t, d = x.shape
    e_num, h_dim, _ = W1.shape
    k = 2
    nt = (t * k) // _BT + e_num
    np_ = nt * _BT
    nc = h_dim // _HC
    x2 = x.reshape(t, d)

    dst0, dst1, g1, g2, te, lb = pl.pallas_call(
        _route_body,
        out_shape=[
            jax.ShapeDtypeStruct((t, 1), jnp.float32),
            jax.ShapeDtypeStruct((t, 1), jnp.float32),
            jax.ShapeDtypeStruct((t, 1), jnp.float32),
            jax.ShapeDtypeStruct((t, 1), jnp.float32),
            jax.ShapeDtypeStruct((1, nt), jnp.int32),
            jax.ShapeDtypeStruct((1, 1), jnp.float32),
        ],
        out_specs=[
            pl.BlockSpec(memory_space=pltpu.VMEM),
            pl.BlockSpec(memory_space=pltpu.VMEM),
            pl.BlockSpec(memory_space=pltpu.VMEM),
            pl.BlockSpec(memory_space=pltpu.VMEM),
            pl.BlockSpec(memory_space=pltpu.VMEM),
            pl.BlockSpec(memory_space=pltpu.SMEM),
        ],
    )(x2, Wr)

    dst0r = dst0.reshape(1, t)
    dst1r = dst1.reshape(1, t)
    te1 = te.reshape(nt)

    xs, gs = pl.pallas_call(
        _gather_body,
        grid=(np_ // _PBS,),
        in_specs=[
            pl.BlockSpec((1, t), lambda i: (0, 0)),
            pl.BlockSpec((1, t), lambda i: (0, 0)),
            pl.BlockSpec((t, 1), lambda i: (0, 0)),
            pl.BlockSpec((t, 1), lambda i: (0, 0)),
            pl.BlockSpec((t, d), lambda i: (0, 0)),
        ],
        out_specs=[
            pl.BlockSpec((_PBS, d), lambda i: (i, 0)),
            pl.BlockSpec((_PBS, 1), lambda i: (i, 0)),
        ],
        out_shape=[
            jax.ShapeDtypeStruct((np_, d), jnp.float32),
            jax.ShapeDtypeStruct((np_, 1), jnp.float32),
        ],
    )(dst0r, dst1r, g1, g2, x2)

    grid_spec = pltpu.PrefetchScalarGridSpec(
        num_scalar_prefetch=1,
        grid=(nc, nt),
        in_specs=[
            pl.BlockSpec((np_, d), lambda c, i, te: (0, 0)),
            pl.BlockSpec((np_, 1), lambda c, i, te: (0, 0)),
            pl.BlockSpec((1, _HC, d), lambda c, i, te: (te[i], c, 0)),
            pl.BlockSpec((1, d, _HC), lambda c, i, te: (te[i], 0, c)),
        ],
        out_specs=pl.BlockSpec((np_, d), lambda c, i, te: (0, 0)),
    )
    ys = pl.pallas_call(
        _ffn_body,
        grid_spec=grid_spec,
        out_shape=jax.ShapeDtypeStruct((np_, d), jnp.float32),
    )(te1, xs, gs, W1, W2)

    out = pl.pallas_call(
        _scatter_body,
        grid=(t // _TBS,),
        in_specs=[
            pl.BlockSpec((_TBS, 1), lambda i: (i, 0)),
            pl.BlockSpec((_TBS, 1), lambda i: (i, 0)),
            pl.BlockSpec((np_, d), lambda i: (0, 0)),
        ],
        out_specs=pl.BlockSpec((_TBS, d), lambda i: (i, 0)),
        out_shape=jax.ShapeDtypeStruct((t, d), jnp.float32),
    )(dst0, dst1, ys)

    return out.reshape(b, t, d), lb[0, 0]


# speedup vs baseline: 1.3419x; 1.3419x over previous
import jax
import jax.numpy as jnp
from jax.experimental import pallas as pl
from jax.experimental.pallas import tpu as pltpu

_BT = 256
_HC = 768
_CB = 512
_PBS = 512
_TBS = 256
_SQRT1_2 = 0.7071067811865476


def _route_body(x_ref, wr_ref, dst0_ref, dst1_ref, g1_ref, g2_ref,
                te_ref, lb_ref):
    t = x_ref.shape[0]
    e_num = wr_ref.shape[0]
    nt = te_ref.shape[1]
    x = x_ref[...]
    wr = wr_ref[...]
    logits = jax.lax.dot_general(x, wr, (((1,), (1,)), ((), ())),
                                 preferred_element_type=jnp.float32)
    m = jnp.max(logits, axis=1, keepdims=True)
    ex = jnp.exp(logits - m)
    probs = ex / jnp.sum(ex, axis=1, keepdims=True)
    lane = jax.lax.broadcasted_iota(jnp.int32, (t, e_num), 1)
    p1 = jnp.max(probs, axis=1, keepdims=True)
    i1 = jnp.min(jnp.where(probs == p1, lane, e_num), axis=1, keepdims=True)
    probs2 = jnp.where(lane == i1, -jnp.inf, probs)
    p2 = jnp.max(probs2, axis=1, keepdims=True)
    i2 = jnp.min(jnp.where(probs2 == p2, lane, e_num), axis=1, keepdims=True)
    s = p1 + p2 + 1e-9
    g1_ref[...] = p1 / s
    g2_ref[...] = p2 / s

    one0 = (lane == i1).astype(jnp.float32)
    one1 = (lane == i2).astype(jnp.float32)

    imp = jnp.sum(probs, axis=0, keepdims=True)
    load = jnp.sum(one0 + one1, axis=0, keepdims=True)
    impn = imp / (jnp.sum(imp) + 1e-9)
    loadn = load / (jnp.sum(load) + 1e-9)
    lb_ref[0, 0] = jnp.sum(impn * loadn) * e_num

    cb = _CB
    tri = (jax.lax.broadcasted_iota(jnp.int32, (cb, cb), 0)
           > jax.lax.broadcasted_iota(jnp.int32, (cb, cb), 1)
           ).astype(jnp.float32)
    run = jnp.zeros((1, e_num), jnp.float32)
    ranks = []
    for oh in (one0, one1):
        for b in range(t // cb):
            blk = oh[b * cb:(b + 1) * cb]
            ranks.append(jax.lax.dot_general(
                tri, blk, (((1,), (0,)), ((), ())),
                preferred_element_type=jnp.float32) + run)
            run = run + jnp.sum(blk, axis=0, keepdims=True)
    nb = t // cb
    rank0 = jnp.concatenate(ranks[:nb], axis=0)
    rank1 = jnp.concatenate(ranks[nb:], axis=0)
    counts = run

    ntiles = jnp.floor((counts + (_BT - 1)) * (1.0 / _BT))
    er = jax.lax.broadcasted_iota(jnp.int32, (e_num, e_num), 0)
    ec = jax.lax.broadcasted_iota(jnp.int32, (e_num, e_num), 1)
    ubelow = (er < ec).astype(jnp.float32)
    tile_start = jax.lax.dot_general(
        ntiles, ubelow, (((1,), (0,)), ((), ())),
        preferred_element_type=jnp.float32)
    base = tile_start * float(_BT)
    dst0_ref[...] = jnp.sum(one0 * (base + rank0), axis=1, keepdims=True)
    dst1_ref[...] = jnp.sum(one1 * (base + rank1), axis=1, keepdims=True)

    ident = (er == ec).astype(jnp.float32)
    ts_col = jax.lax.dot_general(
        ident, tile_start, (((1,), (1,)), ((), ())),
        preferred_element_type=jnp.float32)
    tau = jax.lax.broadcasted_iota(jnp.int32, (e_num, nt), 1
                                   ).astype(jnp.float32)
    te = jnp.sum((ts_col <= tau).astype(jnp.float32), axis=0,
                 keepdims=True) - 1.0
    te_ref[...] = te.astype(jnp.int32)


def _gather_body(dst0r_ref, dst1r_ref, x_ref, xs_ref):
    pb = pl.program_id(0)
    t = x_ref.shape[0]
    p_vals = (jax.lax.broadcasted_iota(jnp.int32, (_PBS, t), 0)
              + pb * _PBS).astype(jnp.float32)
    m0 = (dst0r_ref[...] == p_vals)
    m1 = (dst1r_ref[...] == p_vals)
    msum = (m0 | m1).astype(jnp.float32)
    xs_ref[...] = jax.lax.dot_general(
        msum, x_ref[...], (((1,), (0,)), ((), ())),
        preferred_element_type=jnp.float32)


def _ffn_body(te_ref, xs_ref, w1_ref, w2_ref, ys_ref):
    c = pl.program_id(0)
    i = pl.program_id(1)
    nc = pl.num_programs(0)
    rows = pl.ds(i * _BT, _BT)

    xs = xs_ref[rows, :]
    w1 = w1_ref[0]
    w2 = w2_ref[0]
    h = jax.lax.dot_general(xs, w1, (((1,), (1,)), ((), ())),
                            preferred_element_type=jnp.float32)
    h = 0.5 * h * (1.0 + jax.lax.erf(h * _SQRT1_2))
    y = jax.lax.dot_general(h, w2, (((1,), (1,)), ((), ())),
                            preferred_element_type=jnp.float32)

    @pl.when(c == 0)
    def _init():
        ys_ref[rows, :] = y

    @pl.when(c > 0)
    def _acc():
        ys_ref[rows, :] += y


def _scatter_body(dst0_ref, dst1_ref, g1_ref, g2_ref, ys_ref, out_ref):
    tb = out_ref.shape[0]
    np_ = ys_ref.shape[0]
    p_row = jax.lax.broadcasted_iota(jnp.int32, (tb, np_), 1
                                     ).astype(jnp.float32)
    mg = ((dst0_ref[...] == p_row).astype(jnp.float32) * g1_ref[...]
          + (dst1_ref[...] == p_row).astype(jnp.float32) * g2_ref[...])
    out_ref[...] = jax.lax.dot_general(
        mg, ys_ref[...], (((1,), (0,)), ((), ())),
        preferred_element_type=jnp.float32)


def kernel(x, Wr, W1, W2):
    b, t, d = x.shape
    e_num, h_dim, _ = W1.shape
    k = 2
    nt = (t * k) // _BT + e_num
    np_ = nt * _BT
    nc = h_dim // _HC
    x2 = x.reshape(t, d)

    dst0, dst1, g1, g2, te, lb = pl.pallas_call(
        _route_body,
        out_shape=[
            jax.ShapeDtypeStruct((t, 1), jnp.float32),
            jax.ShapeDtypeStruct((t, 1), jnp.float32),
            jax.ShapeDtypeStruct((t, 1), jnp.float32),
            jax.ShapeDtypeStruct((t, 1), jnp.float32),
            jax.ShapeDtypeStruct((1, nt), jnp.int32),
            jax.ShapeDtypeStruct((1, 1), jnp.float32),
        ],
        out_specs=[
            pl.BlockSpec(memory_space=pltpu.VMEM),
            pl.BlockSpec(memory_space=pltpu.VMEM),
            pl.BlockSpec(memory_space=pltpu.VMEM),
            pl.BlockSpec(memory_space=pltpu.VMEM),
            pl.BlockSpec(memory_space=pltpu.VMEM),
            pl.BlockSpec(memory_space=pltpu.SMEM),
        ],
    )(x2, Wr)

    dst0r = dst0.reshape(1, t)
    dst1r = dst1.reshape(1, t)
    te1 = te.reshape(nt)

    xs = pl.pallas_call(
        _gather_body,
        grid=(np_ // _PBS,),
        in_specs=[
            pl.BlockSpec((1, t), lambda i: (0, 0)),
            pl.BlockSpec((1, t), lambda i: (0, 0)),
            pl.BlockSpec((t, d), lambda i: (0, 0)),
        ],
        out_specs=pl.BlockSpec((_PBS, d), lambda i: (i, 0)),
        out_shape=jax.ShapeDtypeStruct((np_, d), jnp.float32),
    )(dst0r, dst1r, x2)

    grid_spec = pltpu.PrefetchScalarGridSpec(
        num_scalar_prefetch=1,
        grid=(nc, nt),
        in_specs=[
            pl.BlockSpec((np_, d), lambda c, i, te: (0, 0)),
            pl.BlockSpec((1, _HC, d), lambda c, i, te: (te[i], c, 0)),
            pl.BlockSpec((1, d, _HC), lambda c, i, te: (te[i], 0, c)),
        ],
        out_specs=pl.BlockSpec((np_, d), lambda c, i, te: (0, 0)),
    )
    ys = pl.pallas_call(
        _ffn_body,
        grid_spec=grid_spec,
        out_shape=jax.ShapeDtypeStruct((np_, d), jnp.float32),
    )(te1, xs, W1, W2)

    out = pl.pallas_call(
        _scatter_body,
        grid=(t // _TBS,),
        in_specs=[
            pl.BlockSpec((_TBS, 1), lambda i: (i, 0)),
            pl.BlockSpec((_TBS, 1), lambda i: (i, 0)),
            pl.BlockSpec((_TBS, 1), lambda i: (i, 0)),
            pl.BlockSpec((_TBS, 1), lambda i: (i, 0)),
            pl.BlockSpec((np_, d), lambda i: (0, 0)),
        ],
        out_specs=pl.BlockSpec((_TBS, d), lambda i: (i, 0)),
        out_shape=jax.ShapeDtypeStruct((t, d), jnp.float32),
    )(dst0, dst1, g1, g2, ys)

    return out.reshape(b, t, d), lb[0, 0]


# speedup vs baseline: 1.4617x; 1.0893x over previous
import jax
import jax.numpy as jnp
from jax.experimental import pallas as pl
from jax.experimental.pallas import tpu as pltpu

_BT = 256
_HC = 1024
_CB = 512
_PBS = 512
_TBS = 512
_SQRT1_2 = 0.7071067811865476


def _route_body(x_ref, wr_ref, dst0_ref, dst1_ref, g1_ref, g2_ref,
                te_ref, lb_ref):
    t = x_ref.shape[0]
    e_num = wr_ref.shape[0]
    nt = te_ref.shape[1]
    x = x_ref[...]
    wr = wr_ref[...]
    logits = jax.lax.dot_general(x, wr, (((1,), (1,)), ((), ())),
                                 preferred_element_type=jnp.float32)
    m = jnp.max(logits, axis=1, keepdims=True)
    ex = jnp.exp(logits - m)
    probs = ex / jnp.sum(ex, axis=1, keepdims=True)
    lane = jax.lax.broadcasted_iota(jnp.int32, (t, e_num), 1)
    p1 = jnp.max(probs, axis=1, keepdims=True)
    i1 = jnp.min(jnp.where(probs == p1, lane, e_num), axis=1, keepdims=True)
    probs2 = jnp.where(lane == i1, -jnp.inf, probs)
    p2 = jnp.max(probs2, axis=1, keepdims=True)
    i2 = jnp.min(jnp.where(probs2 == p2, lane, e_num), axis=1, keepdims=True)
    s = p1 + p2 + 1e-9
    g1_ref[...] = p1 / s
    g2_ref[...] = p2 / s

    one0 = (lane == i1).astype(jnp.float32)
    one1 = (lane == i2).astype(jnp.float32)

    imp = jnp.sum(probs, axis=0, keepdims=True)
    load = jnp.sum(one0 + one1, axis=0, keepdims=True)
    impn = imp / (jnp.sum(imp) + 1e-9)
    loadn = load / (jnp.sum(load) + 1e-9)
    lb_ref[0, 0] = jnp.sum(impn * loadn) * e_num

    cb = _CB
    tri = (jax.lax.broadcasted_iota(jnp.int32, (cb, cb), 0)
           > jax.lax.broadcasted_iota(jnp.int32, (cb, cb), 1)
           ).astype(jnp.float32)
    run = jnp.zeros((1, e_num), jnp.float32)
    ranks = []
    for oh in (one0, one1):
        for b in range(t // cb):
            blk = oh[b * cb:(b + 1) * cb]
            ranks.append(jax.lax.dot_general(
                tri, blk, (((1,), (0,)), ((), ())),
                preferred_element_type=jnp.float32) + run)
            run = run + jnp.sum(blk, axis=0, keepdims=True)
    nb = t // cb
    rank0 = jnp.concatenate(ranks[:nb], axis=0)
    rank1 = jnp.concatenate(ranks[nb:], axis=0)
    counts = run

    ntiles = jnp.floor((counts + (_BT - 1)) * (1.0 / _BT))
    er = jax.lax.broadcasted_iota(jnp.int32, (e_num, e_num), 0)
    ec = jax.lax.broadcasted_iota(jnp.int32, (e_num, e_num), 1)
    ubelow = (er < ec).astype(jnp.float32)
    tile_start = jax.lax.dot_general(
        ntiles, ubelow, (((1,), (0,)), ((), ())),
        preferred_element_type=jnp.float32)
    base = tile_start * float(_BT)
    dst0_ref[...] = jnp.sum(one0 * (base + rank0), axis=1, keepdims=True)
    dst1_ref[...] = jnp.sum(one1 * (base + rank1), axis=1, keepdims=True)

    ident = (er == ec).astype(jnp.float32)
    ts_col = jax.lax.dot_general(
        ident, tile_start, (((1,), (1,)), ((), ())),
        preferred_element_type=jnp.float32)
    tau = jax.lax.broadcasted_iota(jnp.int32, (e_num, nt), 1
                                   ).astype(jnp.float32)
    te = jnp.sum((ts_col <= tau).astype(jnp.float32), axis=0,
                 keepdims=True) - 1.0
    te_ref[...] = te.astype(jnp.int32)


def _gather_body(dst0r_ref, dst1r_ref, x_ref, xs_ref):
    pb = pl.program_id(0)
    t = x_ref.shape[0]
    p_vals = (jax.lax.broadcasted_iota(jnp.int32, (_PBS, t), 0)
              + pb * _PBS).astype(jnp.float32)
    m0 = (dst0r_ref[...] == p_vals)
    m1 = (dst1r_ref[...] == p_vals)
    msum = (m0 | m1).astype(jnp.float32)
    xs_ref[...] = jax.lax.dot_general(
        msum, x_ref[...], (((1,), (0,)), ((), ())),
        preferred_element_type=jnp.float32)


def _ffn_body(te_ref, xs_ref, w1_ref, w2_ref, ys_ref):
    c = pl.program_id(0)
    i = pl.program_id(1)
    nc = pl.num_programs(0)
    rows = pl.ds(i * _BT, _BT)

    xs = xs_ref[rows, :]
    w1 = w1_ref[0]
    w2 = w2_ref[0]
    h = jax.lax.dot_general(xs, w1, (((1,), (1,)), ((), ())),
                            preferred_element_type=jnp.float32)
    h = 0.5 * h * (1.0 + jax.lax.erf(h * _SQRT1_2))
    y = jax.lax.dot_general(h, w2, (((1,), (1,)), ((), ())),
                            preferred_element_type=jnp.float32)

    @pl.when(c == 0)
    def _init():
        ys_ref[rows, :] = y

    @pl.when(c > 0)
    def _acc():
        ys_ref[rows, :] += y


def _scatter_body(dst0_ref, dst1_ref, g1_ref, g2_ref, ys_ref, out_ref):
    tb = out_ref.shape[0]
    np_ = ys_ref.shape[0]
    p_row = jax.lax.broadcasted_iota(jnp.int32, (tb, np_), 1
                                     ).astype(jnp.float32)
    mg = ((dst0_ref[...] == p_row).astype(jnp.float32) * g1_ref[...]
          + (dst1_ref[...] == p_row).astype(jnp.float32) * g2_ref[...])
    out_ref[...] = jax.lax.dot_general(
        mg, ys_ref[...], (((1,), (0,)), ((), ())),
        preferred_element_type=jnp.float32)


def kernel(x, Wr, W1, W2):
    b, t, d = x.shape
    e_num, h_dim, _ = W1.shape
    k = 2
    nt = (t * k) // _BT + e_num
    np_ = nt * _BT
    nc = h_dim // _HC
    x2 = x.reshape(t, d)

    dst0, dst1, g1, g2, te, lb = pl.pallas_call(
        _route_body,
        out_shape=[
            jax.ShapeDtypeStruct((t, 1), jnp.float32),
            jax.ShapeDtypeStruct((t, 1), jnp.float32),
            jax.ShapeDtypeStruct((t, 1), jnp.float32),
            jax.ShapeDtypeStruct((t, 1), jnp.float32),
            jax.ShapeDtypeStruct((1, nt), jnp.int32),
            jax.ShapeDtypeStruct((1, 1), jnp.float32),
        ],
        out_specs=[
            pl.BlockSpec(memory_space=pltpu.VMEM),
            pl.BlockSpec(memory_space=pltpu.VMEM),
            pl.BlockSpec(memory_space=pltpu.VMEM),
            pl.BlockSpec(memory_space=pltpu.VMEM),
            pl.BlockSpec(memory_space=pltpu.VMEM),
            pl.BlockSpec(memory_space=pltpu.SMEM),
        ],
    )(x2, Wr)

    dst0r = dst0.reshape(1, t)
    dst1r = dst1.reshape(1, t)
    te1 = te.reshape(nt)

    xs = pl.pallas_call(
        _gather_body,
        grid=(np_ // _PBS,),
        in_specs=[
            pl.BlockSpec((1, t), lambda i: (0, 0)),
            pl.BlockSpec((1, t), lambda i: (0, 0)),
            pl.BlockSpec((t, d), lambda i: (0, 0)),
        ],
        out_specs=pl.BlockSpec((_PBS, d), lambda i: (i, 0)),
        out_shape=jax.ShapeDtypeStruct((np_, d), jnp.float32),
    )(dst0r, dst1r, x2)

    grid_spec = pltpu.PrefetchScalarGridSpec(
        num_scalar_prefetch=1,
        grid=(nc, nt),
        in_specs=[
            pl.BlockSpec((np_, d), lambda c, i, te: (0, 0)),
            pl.BlockSpec((1, _HC, d), lambda c, i, te: (te[i], c, 0)),
            pl.BlockSpec((1, d, _HC), lambda c, i, te: (te[i], 0, c)),
        ],
        out_specs=pl.BlockSpec((np_, d), lambda c, i, te: (0, 0)),
    )
    ys = pl.pallas_call(
        _ffn_body,
        grid_spec=grid_spec,
        out_shape=jax.ShapeDtypeStruct((np_, d), jnp.float32),
    )(te1, xs, W1, W2)

    out = pl.pallas_call(
        _scatter_body,
        grid=(t // _TBS,),
        in_specs=[
            pl.BlockSpec((_TBS, 1), lambda i: (i, 0)),
            pl.BlockSpec((_TBS, 1), lambda i: (i, 0)),
            pl.BlockSpec((_TBS, 1), lambda i: (i, 0)),
            pl.BlockSpec((_TBS, 1), lambda i: (i, 0)),
            pl.BlockSpec((np_, d), lambda i: (0, 0)),
        ],
        out_specs=pl.BlockSpec((_TBS, d), lambda i: (i, 0)),
        out_shape=jax.ShapeDtypeStruct((t, d), jnp.float32),
    )(dst0, dst1, g1, g2, ys)

    return out.reshape(b, t, d), lb[0, 0]


# speedup vs baseline: 1.5060x; 1.0303x over previous
import jax
import jax.numpy as jnp
from jax.experimental import pallas as pl
from jax.experimental.pallas import tpu as pltpu

_BT = 256
_HC = 1024
_CB = 512
_PBS = 512
_TBS = 512
_SQRT1_2 = 0.7071067811865476


def _route_body(x_ref, wr_ref, dst0_ref, dst1_ref, g1_ref, g2_ref,
                te_ref, lb_ref):
    t = x_ref.shape[0]
    e_num = wr_ref.shape[0]
    nt = te_ref.shape[1]
    x = x_ref[...]
    wr = wr_ref[...]
    logits = jax.lax.dot_general(x, wr, (((1,), (1,)), ((), ())),
                                 preferred_element_type=jnp.float32)
    m = jnp.max(logits, axis=1, keepdims=True)
    ex = jnp.exp(logits - m)
    probs = ex / jnp.sum(ex, axis=1, keepdims=True)
    lane = jax.lax.broadcasted_iota(jnp.int32, (t, e_num), 1)
    p1 = jnp.max(probs, axis=1, keepdims=True)
    i1 = jnp.min(jnp.where(probs == p1, lane, e_num), axis=1, keepdims=True)
    probs2 = jnp.where(lane == i1, -jnp.inf, probs)
    p2 = jnp.max(probs2, axis=1, keepdims=True)
    i2 = jnp.min(jnp.where(probs2 == p2, lane, e_num), axis=1, keepdims=True)
    s = p1 + p2 + 1e-9
    g1_ref[...] = p1 / s
    g2_ref[...] = p2 / s

    one0 = (lane == i1).astype(jnp.float32)
    one1 = (lane == i2).astype(jnp.float32)

    imp = jnp.sum(probs, axis=0, keepdims=True)
    load = jnp.sum(one0 + one1, axis=0, keepdims=True)
    impn = imp / (jnp.sum(imp) + 1e-9)
    loadn = load / (jnp.sum(load) + 1e-9)
    lb_ref[0, 0] = jnp.sum(impn * loadn) * e_num

    cb = _CB
    tri = (jax.lax.broadcasted_iota(jnp.int32, (cb, cb), 0)
           > jax.lax.broadcasted_iota(jnp.int32, (cb, cb), 1)
           ).astype(jnp.float32)
    run = jnp.zeros((1, e_num), jnp.float32)
    ranks = []
    for oh in (one0, one1):
        for b in range(t // cb):
            blk = oh[b * cb:(b + 1) * cb]
            ranks.append(jax.lax.dot_general(
                tri, blk, (((1,), (0,)), ((), ())),
                preferred_element_type=jnp.float32) + run)
            run = run + jnp.sum(blk, axis=0, keepdims=True)
    nb = t // cb
    rank0 = jnp.concatenate(ranks[:nb], axis=0)
    rank1 = jnp.concatenate(ranks[nb:], axis=0)
    counts = run

    ntiles = jnp.floor((counts + (_BT - 1)) * (1.0 / _BT))
    er = jax.lax.broadcasted_iota(jnp.int32, (e_num, e_num), 0)
    ec = jax.lax.broadcasted_iota(jnp.int32, (e_num, e_num), 1)
    ubelow = (er < ec).astype(jnp.float32)
    tile_start = jax.lax.dot_general(
        ntiles, ubelow, (((1,), (0,)), ((), ())),
        preferred_element_type=jnp.float32)
    base = tile_start * float(_BT)
    dst0_ref[...] = jnp.sum(one0 * (base + rank0), axis=1, keepdims=True)
    dst1_ref[...] = jnp.sum(one1 * (base + rank1), axis=1, keepdims=True)

    ident = (er == ec).astype(jnp.float32)
    ts_col = jax.lax.dot_general(
        ident, tile_start, (((1,), (1,)), ((), ())),
        preferred_element_type=jnp.float32)
    tau = jax.lax.broadcasted_iota(jnp.int32, (e_num, nt), 1
                                   ).astype(jnp.float32)
    te = jnp.sum((ts_col <= tau).astype(jnp.float32), axis=0,
                 keepdims=True) - 1.0
    te_ref[...] = te.astype(jnp.int32)


def _gather_body(dst0r_ref, dst1r_ref, x_ref, xs_ref):
    pb = pl.program_id(0)
    t = x_ref.shape[0]
    p_vals = (jax.lax.broadcasted_iota(jnp.int32, (_PBS, t), 0)
              + pb * _PBS).astype(jnp.float32)
    m0 = (dst0r_ref[...] == p_vals)
    m1 = (dst1r_ref[...] == p_vals)
    msum = (m0 | m1).astype(jnp.float32)
    xs_ref[...] = jax.lax.dot_general(
        msum, x_ref[...], (((1,), (0,)), ((), ())),
        preferred_element_type=jnp.float32).astype(jnp.bfloat16)


def _ffn_body(te_ref, xs_ref, w1_ref, w2_ref, ys_ref, acc_ref):
    c = pl.program_id(0)
    i = pl.program_id(1)
    nc = pl.num_programs(0)
    rows = pl.ds(i * _BT, _BT)

    xs = xs_ref[rows, :].astype(jnp.float32)
    w1 = w1_ref[0]
    w2 = w2_ref[0]
    h = jax.lax.dot_general(xs, w1, (((1,), (1,)), ((), ())),
                            preferred_element_type=jnp.float32)
    h = 0.5 * h * (1.0 + jax.lax.erf(h * _SQRT1_2))
    y = jax.lax.dot_general(h, w2, (((1,), (1,)), ((), ())),
                            preferred_element_type=jnp.float32)

    @pl.when(c == 0)
    def _init():
        acc_ref[rows, :] = y

    @pl.when(c > 0)
    def _acc():
        acc_ref[rows, :] += y

    @pl.when(c == nc - 1)
    def _emit():
        ys_ref[rows, :] = acc_ref[rows, :].astype(jnp.bfloat16)


def _scatter_body(dst0_ref, dst1_ref, g1_ref, g2_ref, ys_ref, out_ref):
    tb = out_ref.shape[0]
    np_ = ys_ref.shape[0]
    p_row = jax.lax.broadcasted_iota(jnp.int32, (tb, np_), 1
                                     ).astype(jnp.float32)
    mg = ((dst0_ref[...] == p_row).astype(jnp.float32) * g1_ref[...]
          + (dst1_ref[...] == p_row).astype(jnp.float32) * g2_ref[...])
    out_ref[...] = jax.lax.dot_general(
        mg.astype(jnp.bfloat16), ys_ref[...], (((1,), (0,)), ((), ())),
        preferred_element_type=jnp.float32)


def kernel(x, Wr, W1, W2):
    b, t, d = x.shape
    e_num, h_dim, _ = W1.shape
    k = 2
    nt = (t * k) // _BT + e_num
    np_ = nt * _BT
    nc = h_dim // _HC
    x2 = x.reshape(t, d)

    dst0, dst1, g1, g2, te, lb = pl.pallas_call(
        _route_body,
        out_shape=[
            jax.ShapeDtypeStruct((t, 1), jnp.float32),
            jax.ShapeDtypeStruct((t, 1), jnp.float32),
            jax.ShapeDtypeStruct((t, 1), jnp.float32),
            jax.ShapeDtypeStruct((t, 1), jnp.float32),
            jax.ShapeDtypeStruct((1, nt), jnp.int32),
            jax.ShapeDtypeStruct((1, 1), jnp.float32),
        ],
        out_specs=[
            pl.BlockSpec(memory_space=pltpu.VMEM),
            pl.BlockSpec(memory_space=pltpu.VMEM),
            pl.BlockSpec(memory_space=pltpu.VMEM),
            pl.BlockSpec(memory_space=pltpu.VMEM),
            pl.BlockSpec(memory_space=pltpu.VMEM),
            pl.BlockSpec(memory_space=pltpu.SMEM),
        ],
    )(x2, Wr)

    dst0r = dst0.reshape(1, t)
    dst1r = dst1.reshape(1, t)
    te1 = te.reshape(nt)

    xs = pl.pallas_call(
        _gather_body,
        grid=(np_ // _PBS,),
        in_specs=[
            pl.BlockSpec((1, t), lambda i: (0, 0)),
            pl.BlockSpec((1, t), lambda i: (0, 0)),
            pl.BlockSpec((t, d), lambda i: (0, 0)),
        ],
        out_specs=pl.BlockSpec((_PBS, d), lambda i: (i, 0)),
        out_shape=jax.ShapeDtypeStruct((np_, d), jnp.bfloat16),
    )(dst0r, dst1r, x2)

    grid_spec = pltpu.PrefetchScalarGridSpec(
        num_scalar_prefetch=1,
        grid=(nc, nt),
        in_specs=[
            pl.BlockSpec((np_, d), lambda c, i, te: (0, 0)),
            pl.BlockSpec((1, _HC, d), lambda c, i, te: (te[i], c, 0)),
            pl.BlockSpec((1, d, _HC), lambda c, i, te: (te[i], 0, c)),
        ],
        out_specs=pl.BlockSpec((np_, d), lambda c, i, te: (0, 0)),
        scratch_shapes=[pltpu.VMEM((np_, d), jnp.float32)],
    )
    ys = pl.pallas_call(
        _ffn_body,
        grid_spec=grid_spec,
        out_shape=jax.ShapeDtypeStruct((np_, d), jnp.bfloat16),
    )(te1, xs, W1, W2)

    out = pl.pallas_call(
        _scatter_body,
        grid=(t // _TBS,),
        in_specs=[
            pl.BlockSpec((_TBS, 1), lambda i: (i, 0)),
            pl.BlockSpec((_TBS, 1), lambda i: (i, 0)),
            pl.BlockSpec((_TBS, 1), lambda i: (i, 0)),
            pl.BlockSpec((_TBS, 1), lambda i: (i, 0)),
            pl.BlockSpec((np_, d), lambda i: (0, 0)),
        ],
        out_specs=pl.BlockSpec((_TBS, d), lambda i: (i, 0)),
        out_shape=jax.ShapeDtypeStruct((t, d), jnp.float32),
    )(dst0, dst1, g1, g2, ys)

    return out.reshape(b, t, d), lb[0, 0]
